# Initial kernel scaffold; baseline (speedup 1.0000x reference)
import jax
import jax.numpy as jnp
from jax.experimental import pallas as pl


def kernel(x, Wg, bg, Wn, bn, We_gate, We_up, We_down, Ws_gate, Ws_up, Ws_down):
    raise NotImplementedError("write your pallas kernel here")


# speedup vs baseline: 1.3517x; 1.3517x over previous
import jax
import jax.numpy as jnp
from jax.experimental import pallas as pl
from jax.experimental.pallas import tpu as pltpu


def _moe_body(x_ref, wg_ref, bg_ref, wn_ref, bn_ref, noise_ref,
              weg_ref, weu_ref, wed_ref, wsg_ref, wsu_ref, wsd_ref,
              out_ref, gate_ref, *, E, NH):
    e = pl.program_id(0)
    h = pl.program_id(1)
    xr = x_ref[...]
    T = xr.shape[0]

    @pl.when(jnp.logical_and(e == 0, h == 0))
    def _router():
        logits = jnp.dot(xr, wg_ref[...], preferred_element_type=jnp.float32) + bg_ref[...]
        nlog = jnp.dot(xr, wn_ref[...], preferred_element_type=jnp.float32) + bn_ref[...]
        sp = jnp.maximum(nlog, 0.0) + jnp.log1p(jnp.exp(-jnp.abs(nlog)))
        noisy = logits + noise_ref[...] * sp
        idx = jax.lax.broadcasted_iota(jnp.int32, noisy.shape, 1)
        m1 = jnp.max(noisy, axis=1, keepdims=True)
        am1 = jnp.min(jnp.where(noisy == m1, idx, E), axis=1, keepdims=True)
        n2 = jnp.where(idx == am1, -jnp.inf, noisy)
        m2 = jnp.max(n2, axis=1, keepdims=True)
        am2 = jnp.min(jnp.where(n2 == m2, idx, E), axis=1, keepdims=True)
        mask = jnp.logical_or(idx == am1, idx == am2)
        ex = jnp.where(mask, jnp.exp(noisy - m1), 0.0)
        gate_ref[...] = ex / jnp.sum(ex, axis=1, keepdims=True)
        out_ref[...] = jnp.zeros_like(out_ref)

    routed = e < E
    wg = jnp.where(routed, weg_ref[0], wsg_ref[0])
    wu = jnp.where(routed, weu_ref[0], wsu_ref[0])
    wd = jnp.where(routed, wed_ref[0], wsd_ref[0])
    g = jnp.dot(xr, wg, preferred_element_type=jnp.float32)
    u = jnp.dot(xr, wu, preferred_element_type=jnp.float32)
    act = (g / (1.0 + jnp.exp(-g))) * u
    eo = (jax.lax.broadcasted_iota(jnp.int32, (E, 1), 0) == e).astype(jnp.float32)
    gcol = jnp.dot(gate_ref[...], eo, preferred_element_type=jnp.float32)
    w = jnp.where(routed, gcol, jnp.ones_like(gcol))
    out_ref[...] += jnp.dot(act * w, wd, preferred_element_type=jnp.float32)


def kernel(x, Wg, bg, Wn, bn, We_gate, We_up, We_down, Ws_gate, Ws_up, Ws_down):
    B, S, D = x.shape
    E, _, H = We_gate.shape
    NS = Ws_gate.shape[0]
    T = B * S
    xf = x.reshape(T, D)
    noise = jax.random.normal(jax.random.key(42), (B, S, E), jnp.float32).reshape(T, E)
    BH = 512
    NH = H // BH
    grid = (E + NS, NH)

    def we_map(e, h):
        return (jnp.minimum(e, E - 1), 0, jnp.where(e < E, h, NH - 1))

    def wed_map(e, h):
        return (jnp.minimum(e, E - 1), jnp.where(e < E, h, NH - 1), 0)

    def ws_map(e, h):
        return (jnp.maximum(e - E, 0), 0, jnp.where(e < E, 0, h))

    def wsd_map(e, h):
        return (jnp.maximum(e - E, 0), jnp.where(e < E, 0, h), 0)

    const2 = lambda e, h: (0, 0)
    import functools
    body = functools.partial(_moe_body, E=E, NH=NH)
    out = pl.pallas_call(
        body,
        grid=grid,
        in_specs=[
            pl.BlockSpec((T, D), const2),
            pl.BlockSpec((D, E), const2),
            pl.BlockSpec((1, E), const2),
            pl.BlockSpec((D, E), const2),
            pl.BlockSpec((1, E), const2),
            pl.BlockSpec((T, E), const2),
            pl.BlockSpec((1, D, BH), we_map),
            pl.BlockSpec((1, D, BH), we_map),
            pl.BlockSpec((1, BH, D), wed_map),
            pl.BlockSpec((1, D, BH), ws_map),
            pl.BlockSpec((1, D, BH), ws_map),
            pl.BlockSpec((1, BH, D), wsd_map),
        ],
        out_specs=pl.BlockSpec((T, D), const2),
        out_shape=jax.ShapeDtypeStruct((T, D), jnp.float32),
        scratch_shapes=[pltpu.VMEM((T, E), jnp.float32)],
        compiler_params=pltpu.CompilerParams(
            dimension_semantics=("arbitrary", "arbitrary")),
    )(xf, Wg, bg.reshape(1, E), Wn, bn.reshape(1, E), noise,
      We_gate, We_up, We_down, Ws_gate, Ws_up, Ws_down)
    return out.reshape(B, S, D)
